# baseline (speedup 1.0000x reference)
import functools
import jax
import jax.numpy as jnp
from jax import lax
from jax.experimental import pallas as pl
from jax.experimental.pallas import tpu as pltpu
from jax.experimental.pallas import tpu_sc as plsc

_B, _D, _N, _KP1 = 1024, 128, 100000, 1024
_T = 0.07
_MOM = 0.5
_NW = 32
_SPW = _B // _NW
_RPW = _N // _NW
_LANES = 16
_DV = _D // _LANES
_WPB = 512
_NBLK = (_N + 2 * _WPB - 1) // (2 * _WPB)
_NWORD = _NBLK * _WPB


def _tc_body(ab_ref, l_ref, mllo, mlhi, mablo, mabhi, pab_ref, plv_ref):
    def pack(q, lo_ref, hi_ref):
        dn = (((1,), (1,)), ((), ()))
        rlo = lax.dot_general(q, lo_ref[...], dn,
                              preferred_element_type=jnp.float32)
        rhi = lax.dot_general(q, hi_ref[...], dn,
                              preferred_element_type=jnp.float32)
        ulo = lax.bitcast_convert_type(rlo, jnp.uint32)
        uhi = lax.bitcast_convert_type(rhi, jnp.uint32)
        ulo = (ulo + 0x7FFF + ((ulo >> 16) & 1)) >> 16
        uhi = (uhi + 0x7FFF + ((uhi >> 16) & 1)) >> 16
        return lax.bitcast_convert_type(ulo | (uhi << 16), jnp.int32)

    q_ab = ab_ref[...]
    q_l = l_ref[...]
    pab_ref[...] = pack(q_ab, mllo, mlhi)
    plv_ref[...] = pack(q_l, mablo, mabhi)


def _rsqrt16(s):
    half = s * 0.5
    i = plsc.bitcast(s, jnp.int32)
    i = jnp.int32(0x5F3759DF) - lax.shift_right_arithmetic(i, 1)
    y = plsc.bitcast(i, jnp.float32)
    for _ in range(4):
        y = y * (1.5 - half * y * y)
    return y


def _body(l_hbm, ab_hbm, y_hbm, idx_hbm, ml_hbm, mab_hbm, pab_hbm, plv_hbm,
          outl_hbm, outab_hbm, nml_hbm, nmab_hbm,
          idx_v, s0, s1, outv, y_v, rowm, rowx, rowu,
          sb0, sb1, scp, sst):
    wid = lax.axis_index("s") * 2 + lax.axis_index("c")
    r0 = wid * _RPW

    cpA = pltpu.async_copy(ml_hbm.at[pl.ds(r0, _RPW)],
                           nml_hbm.at[pl.ds(r0, _RPW)], scp)
    cpB = pltpu.async_copy(mab_hbm.at[pl.ds(r0, _RPW)],
                           nmab_hbm.at[pl.ds(r0, _RPW)], scp)

    lane_iota = lax.iota(jnp.int32, _LANES)

    def extract(slot, row):
        @plsc.parallel_loop(0, _KP1 // _LANES, step=1, unroll=4)
        def grp(g):
            n16 = idx_v[pl.ds(g * _LANES, _LANES)]
            g16 = n16 >> 10
            o16 = jnp.bitwise_and(n16, 1023)
            w16 = (g16 << 9) + jnp.bitwise_and(o16, 511)
            sh = (o16 >> 9) << 4
            w = plsc.load_gather(slot, [w16])
            bits = (w >> sh) << 16
            outv[row, pl.ds(g * _LANES, _LANES)] = (
                plsc.bitcast(bits, jnp.float32) * (1.0 / _T))

    def sample_loop(si, carry):
        b = wid * _SPW + si
        pltpu.async_copy(idx_hbm.at[b], idx_v, sst)
        pltpu.async_copy(pab_hbm.at[b], s0, sb0)
        pltpu.async_copy(plv_hbm.at[b], s1, sb1)
        pltpu.make_async_copy(idx_hbm.at[b], idx_v, sst).wait()
        pltpu.make_async_copy(pab_hbm.at[b], s0, sb0).wait()
        extract(s0, 0)
        pltpu.make_async_copy(plv_hbm.at[b], s1, sb1).wait()
        extract(s1, 1)
        pltpu.sync_copy(outv.at[0], outab_hbm.at[b])
        pltpu.sync_copy(outv.at[1], outl_hbm.at[b])
        return carry

    lax.fori_loop(0, _SPW, sample_loop, 0)

    cpA.wait()
    cpB.wait()
    pltpu.sync_copy(y_hbm, y_v)

    def upd_one(mem_hbm, nmem_hbm, x_hbm, b, y_b):
        pltpu.async_copy(mem_hbm.at[pl.ds(y_b, 1)], rowm, sst)
        pltpu.async_copy(x_hbm.at[pl.ds(b, 1)], rowx, sst)
        pltpu.make_async_copy(mem_hbm.at[pl.ds(y_b, 1)], rowm, sst).wait()
        pltpu.make_async_copy(x_hbm.at[pl.ds(b, 1)], rowx, sst).wait()
        acc = jnp.zeros((_LANES,), jnp.float32)
        for j in range(_DV):
            pos = (rowm[0, pl.ds(j * _LANES, _LANES)] * _MOM
                   + rowx[0, pl.ds(j * _LANES, _LANES)] * (1.0 - _MOM))
            rowu[0, pl.ds(j * _LANES, _LANES)] = pos
            acc = acc + pos * pos
        s = jnp.sum(acc)
        inv = _rsqrt16(jnp.broadcast_to(s, (_LANES,)))
        for j in range(_DV):
            rowu[0, pl.ds(j * _LANES, _LANES)] = (
                rowu[0, pl.ds(j * _LANES, _LANES)] * inv)
        pltpu.sync_copy(rowu, nmem_hbm.at[pl.ds(y_b, 1)])

    def b_loop(b, carry):
        base = jnp.bitwise_and(b, jnp.int32(~15))
        lane = jnp.bitwise_and(b, jnp.int32(15))
        y16 = y_v[pl.ds(base, _LANES)]
        y_b = jnp.sum(jnp.where(lane_iota == lane, y16, 0))

        @pl.when(jnp.logical_and(y_b >= r0, y_b < r0 + _RPW))
        def _():
            upd_one(ml_hbm, nml_hbm, l_hbm, b, y_b)
            upd_one(mab_hbm, nmab_hbm, ab_hbm, b, y_b)

        return carry

    lax.fori_loop(0, _B, b_loop, 0)


def kernel(l, ab, y, idx, memory_l, memory_ab):
    tc = pl.pallas_call(
        _tc_body,
        grid=(_NBLK,),
        in_specs=[
            pl.BlockSpec((_B, _D), lambda j: (0, 0)),
            pl.BlockSpec((_B, _D), lambda j: (0, 0)),
            pl.BlockSpec((_WPB, _D), lambda j: (2 * j, 0)),
            pl.BlockSpec((_WPB, _D), lambda j: (2 * j + 1, 0)),
            pl.BlockSpec((_WPB, _D), lambda j: (2 * j, 0)),
            pl.BlockSpec((_WPB, _D), lambda j: (2 * j + 1, 0)),
        ],
        out_specs=[
            pl.BlockSpec((_B, _WPB), lambda j: (0, j)),
            pl.BlockSpec((_B, _WPB), lambda j: (0, j)),
        ],
        out_shape=[
            jax.ShapeDtypeStruct((_B, _NWORD), jnp.int32),
            jax.ShapeDtypeStruct((_B, _NWORD), jnp.int32),
        ],
    )
    pab_w, plv_w = tc(ab, l, memory_l, memory_l, memory_ab, memory_ab)

    mesh = plsc.VectorSubcoreMesh(core_axis_name="c", subcore_axis_name="s",
                                  num_cores=2, num_subcores=16)
    out_type = [
        jax.ShapeDtypeStruct((_B, _KP1), jnp.float32),
        jax.ShapeDtypeStruct((_B, _KP1), jnp.float32),
        jax.ShapeDtypeStruct((_N, _D), jnp.float32),
        jax.ShapeDtypeStruct((_N, _D), jnp.float32),
    ]
    scratch = [
        pltpu.VMEM((_KP1,), jnp.int32),
        pltpu.VMEM((_NWORD,), jnp.int32),
        pltpu.VMEM((_NWORD,), jnp.int32),
        pltpu.VMEM((2, _KP1), jnp.float32),
        pltpu.VMEM((_B,), jnp.int32),
        pltpu.VMEM((1, _D), jnp.float32),
        pltpu.VMEM((1, _D), jnp.float32),
        pltpu.VMEM((1, _D), jnp.float32),
        pltpu.SemaphoreType.DMA,
        pltpu.SemaphoreType.DMA,
        pltpu.SemaphoreType.DMA,
        pltpu.SemaphoreType.DMA,
    ]
    run = pl.kernel(_body, out_type=out_type, mesh=mesh,
                    scratch_types=scratch,
                    compiler_params=pltpu.CompilerParams(
                        use_tc_tiling_on_sc=False,
                        needs_layout_passes=False))
    out_l2, out_ab2, nml, nmab = run(l, ab, y.astype(jnp.int32), idx,
                                     memory_l, memory_ab, pab_w, plv_w)
    return (out_l2[..., None], out_ab2[..., None], nml, nmab)


# speedup vs baseline: 1.1555x; 1.1555x over previous
import functools
import jax
import jax.numpy as jnp
from jax import lax
from jax.experimental import pallas as pl
from jax.experimental.pallas import tpu as pltpu
from jax.experimental.pallas import tpu_sc as plsc

_B, _D, _N, _KP1 = 1024, 128, 100000, 1024
_T = 0.07
_MOM = 0.5
_NW = 32
_SPW = _B // _NW
_RPW = _N // _NW
_CHUNK = 128
_NCH = _KP1 // _CHUNK
_LANES = 16
_DV = _D // _LANES


def _rsqrt16(s):
    half = s * 0.5
    i = plsc.bitcast(s, jnp.int32)
    i = jnp.int32(0x5F3759DF) - lax.shift_right_arithmetic(i, 1)
    y = plsc.bitcast(i, jnp.float32)
    for _ in range(4):
        y = y * (1.5 - half * y * y)
    return y


def _body(l_hbm, ab_hbm, y_hbm, idx_hbm, ml_hbm, mab_hbm,
          outl_hbm, outab_hbm, nml_hbm, nmab_hbm,
          idx_v, qv, bufl0, bufab0, bufl1, bufab1, outv, y_v,
          rowm, rowx, rowu,
          sl0, sa0, sl1, sa1, scp, sst):
    wid = lax.axis_index("s") * 2 + lax.axis_index("c")
    r0 = wid * _RPW

    cpA = pltpu.async_copy(ml_hbm.at[pl.ds(r0, _RPW)],
                           nml_hbm.at[pl.ds(r0, _RPW)], scp)
    cpB = pltpu.async_copy(mab_hbm.at[pl.ds(r0, _RPW)],
                           nmab_hbm.at[pl.ds(r0, _RPW)], scp)

    lane_iota = lax.iota(jnp.int32, _LANES)
    rings = [(bufl0, bufab0, sl0, sa0), (bufl1, bufab1, sl1, sa1)]

    def issue(c, which):
        bl, ba, sl, sa = rings[which]
        idx_sl = idx_v.at[pl.ds(c * _CHUNK, _CHUNK)]
        pltpu.async_copy(ml_hbm.at[idx_sl], bl, sl)
        pltpu.async_copy(mab_hbm.at[idx_sl], ba, sa)

    def wait(which):
        bl, ba, sl, sa = rings[which]
        pltpu.make_async_copy(ml_hbm.at[pl.ds(0, _CHUNK)], bl, sl).wait()
        pltpu.make_async_copy(mab_hbm.at[pl.ds(0, _CHUNK)], ba, sa).wait()

    def compute(c, bl, ba, qab, ql):
        @plsc.parallel_loop(0, _CHUNK // _LANES, step=1, unroll=1)
        def rowgrp(g):
            k0 = g * _LANES
            acc0 = jnp.zeros((_LANES,), jnp.float32)
            acc1 = jnp.zeros((_LANES,), jnp.float32)
            for r in range(_LANES):
                k = k0 + r
                a0 = bl[k, pl.ds(0, _LANES)] * qab[0]
                a1 = ba[k, pl.ds(0, _LANES)] * ql[0]
                for j in range(1, _DV):
                    a0 = a0 + bl[k, pl.ds(j * _LANES, _LANES)] * qab[j]
                    a1 = a1 + ba[k, pl.ds(j * _LANES, _LANES)] * ql[j]
                acc0 = jnp.where(lane_iota == r, jnp.sum(a0), acc0)
                acc1 = jnp.where(lane_iota == r, jnp.sum(a1), acc1)
            outv[0, pl.ds(c * _CHUNK + k0, _LANES)] = acc0
            outv[1, pl.ds(c * _CHUNK + k0, _LANES)] = acc1

    def sample_loop(si, carry):
        b = wid * _SPW + si
        pltpu.async_copy(idx_hbm.at[b], idx_v, sst)
        pltpu.async_copy(ab_hbm.at[pl.ds(b, 1)], qv.at[pl.ds(0, 1)], sst)
        pltpu.async_copy(l_hbm.at[pl.ds(b, 1)], qv.at[pl.ds(1, 1)], sst)
        pltpu.make_async_copy(idx_hbm.at[b], idx_v, sst).wait()
        pltpu.make_async_copy(ab_hbm.at[pl.ds(b, 1)], qv.at[pl.ds(0, 1)],
                              sst).wait()
        pltpu.make_async_copy(l_hbm.at[pl.ds(b, 1)], qv.at[pl.ds(1, 1)],
                              sst).wait()
        for j in range(2 * _DV):
            qv[j // _DV, pl.ds((j % _DV) * _LANES, _LANES)] = (
                qv[j // _DV, pl.ds((j % _DV) * _LANES, _LANES)] * (1.0 / _T))
        qab = [qv[0, pl.ds(j * _LANES, _LANES)] for j in range(_DV)]
        ql = [qv[1, pl.ds(j * _LANES, _LANES)] for j in range(_DV)]

        issue(0, 0)
        issue(1, 1)

        def pair_loop(g, carry2):
            c0 = 2 * g
            wait(0)
            compute(c0, bufl0, bufab0, qab, ql)

            @pl.when(g < _NCH // 2 - 1)
            def _():
                issue(c0 + 2, 0)

            wait(1)
            compute(c0 + 1, bufl1, bufab1, qab, ql)

            @pl.when(g < _NCH // 2 - 1)
            def _():
                issue(c0 + 3, 1)

            return carry2

        lax.fori_loop(0, _NCH // 2, pair_loop, 0)
        pltpu.sync_copy(outv.at[0], outab_hbm.at[b])
        pltpu.sync_copy(outv.at[1], outl_hbm.at[b])
        return carry

    lax.fori_loop(0, _SPW, sample_loop, 0)

    cpA.wait()
    cpB.wait()
    pltpu.sync_copy(y_hbm, y_v)

    def upd_one(mem_hbm, nmem_hbm, x_hbm, b, y_b):
        pltpu.async_copy(mem_hbm.at[pl.ds(y_b, 1)], rowm, sst)
        pltpu.async_copy(x_hbm.at[pl.ds(b, 1)], rowx, sst)
        pltpu.make_async_copy(mem_hbm.at[pl.ds(y_b, 1)], rowm, sst).wait()
        pltpu.make_async_copy(x_hbm.at[pl.ds(b, 1)], rowx, sst).wait()
        acc = jnp.zeros((_LANES,), jnp.float32)
        for j in range(_DV):
            pos = (rowm[0, pl.ds(j * _LANES, _LANES)] * _MOM
                   + rowx[0, pl.ds(j * _LANES, _LANES)] * (1.0 - _MOM))
            rowu[0, pl.ds(j * _LANES, _LANES)] = pos
            acc = acc + pos * pos
        s = jnp.sum(acc)
        inv = _rsqrt16(jnp.broadcast_to(s, (_LANES,)))
        for j in range(_DV):
            rowu[0, pl.ds(j * _LANES, _LANES)] = (
                rowu[0, pl.ds(j * _LANES, _LANES)] * inv)
        pltpu.sync_copy(rowu, nmem_hbm.at[pl.ds(y_b, 1)])

    def b_loop(b, carry):
        base = jnp.bitwise_and(b, jnp.int32(~15))
        lane = jnp.bitwise_and(b, jnp.int32(15))
        y16 = y_v[pl.ds(base, _LANES)]
        y_b = jnp.sum(jnp.where(lane_iota == lane, y16, 0))

        @pl.when(jnp.logical_and(y_b >= r0, y_b < r0 + _RPW))
        def _():
            upd_one(ml_hbm, nml_hbm, l_hbm, b, y_b)
            upd_one(mab_hbm, nmab_hbm, ab_hbm, b, y_b)

        return carry

    lax.fori_loop(0, _B, b_loop, 0)


def kernel(l, ab, y, idx, memory_l, memory_ab):
    mesh = plsc.VectorSubcoreMesh(core_axis_name="c", subcore_axis_name="s",
                                  num_cores=2, num_subcores=16)
    out_type = [
        jax.ShapeDtypeStruct((_B, _KP1), jnp.float32),
        jax.ShapeDtypeStruct((_B, _KP1), jnp.float32),
        jax.ShapeDtypeStruct((_N, _D), jnp.float32),
        jax.ShapeDtypeStruct((_N, _D), jnp.float32),
    ]
    scratch = [
        pltpu.VMEM((_KP1,), jnp.int32),
        pltpu.VMEM((2, _D), jnp.float32),
        pltpu.VMEM((_CHUNK, _D), jnp.float32),
        pltpu.VMEM((_CHUNK, _D), jnp.float32),
        pltpu.VMEM((_CHUNK, _D), jnp.float32),
        pltpu.VMEM((_CHUNK, _D), jnp.float32),
        pltpu.VMEM((2, _KP1), jnp.float32),
        pltpu.VMEM((_B,), jnp.int32),
        pltpu.VMEM((1, _D), jnp.float32),
        pltpu.VMEM((1, _D), jnp.float32),
        pltpu.VMEM((1, _D), jnp.float32),
        pltpu.SemaphoreType.DMA,
        pltpu.SemaphoreType.DMA,
        pltpu.SemaphoreType.DMA,
        pltpu.SemaphoreType.DMA,
        pltpu.SemaphoreType.DMA,
        pltpu.SemaphoreType.DMA,
    ]
    run = pl.kernel(_body, out_type=out_type, mesh=mesh,
                    scratch_types=scratch,
                    compiler_params=pltpu.CompilerParams(
                        use_tc_tiling_on_sc=False,
                        needs_layout_passes=False))
    out_l2, out_ab2, nml, nmab = run(l, ab, y.astype(jnp.int32), idx,
                                     memory_l, memory_ab)
    return (out_l2[..., None], out_ab2[..., None], nml, nmab)
